# baseline (speedup 1.0000x reference)
import functools
import math

import jax
import jax.numpy as jnp
from jax import lax
from jax.experimental import pallas as pl
from jax.experimental.pallas import tpu as pltpu
from jax.experimental.pallas import tpu_sc as plsc

_SCALE = math.sqrt(512.0)
_BB = 128
_CA = 504
_CB = 496
_DTA = _CA // 8
_DTB = _CB // 8


def _scale_body(t_ref, o_ref):
    o_ref[...] = t_ref[...] * _SCALE


def _scale_table(table):
    return pl.pallas_call(
        _scale_body,
        out_shape=jax.ShapeDtypeStruct(table.shape, table.dtype),
    )(table)


@functools.cache
def _make_gather(BN, S, V, D):
    info = plsc.get_sparse_core_info()
    nc, ns, nl = info.num_cores, info.num_subcores, info.num_lanes
    nw = nc * ns
    n_btiles = BN // _BB
    n_blocks = S * n_btiles
    bpw = n_blocks // nw
    n_bpairs = bpw // 2
    mesh = plsc.VectorSubcoreMesh(core_axis_name="c", subcore_axis_name="s")

    @functools.partial(
        pl.kernel,
        out_type=jax.ShapeDtypeStruct((S, D // 8, n_btiles, 8, _BB),
                                      jnp.float32),
        mesh=mesh,
        scratch_types=[
            pltpu.VMEM((2, _BB), jnp.int32),
            pltpu.VMEM((_BB, _CA), jnp.float32),
            pltpu.VMEM((_BB, _CB), jnp.float32),
            pltpu.VMEM((8, _BB), jnp.float32),
            pltpu.SemaphoreType.DMA,
            pltpu.SemaphoreType.DMA,
            pltpu.SemaphoreType.DMA,
            pltpu.SemaphoreType.DMA,
        ],
        compiler_params=pltpu.CompilerParams(use_tc_tiling_on_sc=False,
                                             needs_layout_passes=False,
                                             disable_bounds_checks=True),
    )
    def gather(xt_hbm, tbl_a, tbl_b, out_hbm,
               idx_v, r_a, r_b, c0,
               gs_a, gs_b, os0, isem):
        cid = lax.axis_index("c")
        sid = lax.axis_index("s")
        wid = sid * nc + cid
        g_base = wid * bpw
        chunk = (c0, c0)
        osem = (os0, os0)

        def sbt(blk):
            g = g_base + blk
            return lax.div(g, n_btiles), lax.rem(g, n_btiles)

        def idx_src(blk):
            s, bt = sbt(blk)
            return xt_hbm.at[s, pl.ds(bt * _BB, _BB)]

        def load_idx(p, blk):
            pltpu.async_copy(idx_src(blk), idx_v.at[p], isem)

        def wait_idx(p, blk):
            pltpu.make_async_copy(idx_src(blk), idx_v.at[p], isem).wait()

        def start_ga(p):
            pltpu.async_copy(tbl_a.at[idx_v.at[p]], r_a, gs_a)

        def wait_ga(p):
            pltpu.make_async_copy(tbl_a.at[idx_v.at[p]], r_a, gs_a).wait()

        def start_gb(p):
            pltpu.async_copy(tbl_b.at[idx_v.at[p]], r_b, gs_b)

        def wait_gb(p):
            pltpu.make_async_copy(tbl_b.at[idx_v.at[p]], r_b, gs_b).wait()

        def wait_out(p):
            pltpu.make_async_copy(chunk[p], out_hbm.at[0, 0, 0],
                                  osem[p]).wait()

        base_vecs = [
            (lax.iota(jnp.int32, nl) + g * nl) for g in range(8)
        ]

        def do_dt(rb, d, dtg, s, bt, p):
            wait_out(p)
            cb = chunk[p]
            w8 = jnp.full((nl,), 0, jnp.int32) + d * 8
            for ds in range(8):
                wv = w8 + ds
                for gi in range(8):
                    cb[ds, pl.ds(gi * nl, nl)] = plsc.load_gather(
                        rb, [base_vecs[gi], wv])
            pltpu.async_copy(cb, out_hbm.at[s, dtg, bt], osem[p])

        def half_a(s, bt):
            def pair(j):
                do_dt(r_a, 2 * j, 2 * j, s, bt, 0)
                do_dt(r_a, 2 * j + 1, 2 * j + 1, s, bt, 1)

            lax.fori_loop(0, (_DTA - 1) // 2,
                          lambda j, c: (pair(j), c)[1], 0, unroll=False)
            do_dt(r_a, _DTA - 1, _DTA - 1, s, bt, 0)

        def half_b(s, bt):
            def pair(j):
                do_dt(r_b, 2 * j, _DTA + 2 * j, s, bt, 1)
                do_dt(r_b, 2 * j + 1, _DTA + 2 * j + 1, s, bt, 0)

            lax.fori_loop(0, _DTB // 2,
                          lambda j, c: (pair(j), c)[1], 0, unroll=False)

        def emit_block(n, not_last):
            h = lax.rem(n, 2)
            s, bt = sbt(n)
            wait_ga(h)
            @pl.when(not_last)
            def _():
                load_idx(1 - h, n + 1)
            half_a(s, bt)
            @pl.when(not_last)
            def _():
                wait_idx(1 - h, n + 1)
                start_ga(1 - h)
            wait_gb(h)
            half_b(s, bt)
            @pl.when(not_last)
            def _():
                start_gb(1 - h)

        load_idx(0, 0)
        wait_idx(0, 0)
        start_ga(0)
        start_gb(0)
        s0, bt0 = sbt(0)
        pltpu.async_copy(chunk[0], out_hbm.at[s0, 0, bt0], osem[0])

        def block_body(n, carry):
            emit_block(n, n < bpw - 1)
            return carry

        lax.fori_loop(0, bpw, block_body, 0, unroll=False)

        wait_out(0)

    return gather


def kernel(x, table):
    BN, S = x.shape
    V, D = table.shape
    scaled = _scale_table(table)
    tbl_a = scaled[:, :_CA]
    tbl_b = scaled[:, _CA:]
    xt = x.T.astype(jnp.int32)
    o5 = _make_gather(BN, S, V, D)(xt, tbl_a, tbl_b)
    return jnp.transpose(o5, (2, 4, 0, 1, 3)).reshape(BN, S, D)


# speedup vs baseline: 1.2183x; 1.2183x over previous
import functools
import math

import jax
import jax.numpy as jnp
from jax import lax
from jax.experimental import pallas as pl
from jax.experimental.pallas import tpu as pltpu
from jax.experimental.pallas import tpu_sc as plsc

_SCALE = math.sqrt(512.0)
_BB = 128
_CA = 504
_CB = 496
_DTA = _CA // 8
_DTB = _CB // 8


def _scale_body(t_ref, o_ref):
    o_ref[...] = t_ref[...] * _SCALE


def _scale_table(table):
    return pl.pallas_call(
        _scale_body,
        out_shape=jax.ShapeDtypeStruct(table.shape, table.dtype),
    )(table)


@functools.cache
def _make_gather(BN, S, V, D):
    info = plsc.get_sparse_core_info()
    nc, ns, nl = info.num_cores, info.num_subcores, info.num_lanes
    nw = nc * ns
    n_btiles = BN // _BB
    n_blocks = S * n_btiles
    bpw = n_blocks // nw
    n_bpairs = bpw // 2
    mesh = plsc.VectorSubcoreMesh(core_axis_name="c", subcore_axis_name="s")

    @functools.partial(
        pl.kernel,
        out_type=jax.ShapeDtypeStruct((S, D // 8, n_btiles, 8, _BB),
                                      jnp.float32),
        mesh=mesh,
        scratch_types=[
            pltpu.VMEM((2, _BB), jnp.int32),
            pltpu.VMEM((_BB, _CA), jnp.float32),
            pltpu.VMEM((_BB, _CB), jnp.float32),
            pltpu.VMEM((8, _BB), jnp.float32),
            pltpu.VMEM((8, _BB), jnp.float32),
            pltpu.SemaphoreType.DMA,
            pltpu.SemaphoreType.DMA,
            pltpu.SemaphoreType.DMA,
            pltpu.SemaphoreType.DMA,
            pltpu.SemaphoreType.DMA,
        ],
        compiler_params=pltpu.CompilerParams(use_tc_tiling_on_sc=False,
                                             needs_layout_passes=False,
                                             disable_bounds_checks=True),
    )
    def gather(xt_hbm, tbl_a, tbl_b, out_hbm,
               idx_v, r_a, r_b, c0, c1,
               gs_a, gs_b, os0, os1, isem):
        cid = lax.axis_index("c")
        sid = lax.axis_index("s")
        wid = sid * nc + cid
        g_base = wid * bpw
        chunk = (c0, c1)
        osem = (os0, os1)

        def sbt(blk):
            g = g_base + blk
            return lax.div(g, n_btiles), lax.rem(g, n_btiles)

        def idx_src(blk):
            s, bt = sbt(blk)
            return xt_hbm.at[s, pl.ds(bt * _BB, _BB)]

        def load_idx(p, blk):
            pltpu.async_copy(idx_src(blk), idx_v.at[p], isem)

        def wait_idx(p, blk):
            pltpu.make_async_copy(idx_src(blk), idx_v.at[p], isem).wait()

        def start_ga(p):
            pltpu.async_copy(tbl_a.at[idx_v.at[p]], r_a, gs_a)

        def wait_ga(p):
            pltpu.make_async_copy(tbl_a.at[idx_v.at[p]], r_a, gs_a).wait()

        def start_gb(p):
            pltpu.async_copy(tbl_b.at[idx_v.at[p]], r_b, gs_b)

        def wait_gb(p):
            pltpu.make_async_copy(tbl_b.at[idx_v.at[p]], r_b, gs_b).wait()

        def wait_out(p):
            pltpu.make_async_copy(chunk[p], out_hbm.at[0, 0, 0],
                                  osem[p]).wait()

        base_vecs = [
            (lax.iota(jnp.int32, nl) + g * nl) for g in range(8)
        ]

        def do_dt(rb, d, dtg, s, bt, p):
            wait_out(p)
            cb = chunk[p]
            w8 = jnp.full((nl,), 0, jnp.int32) + d * 8

            def ds_body(ds, carry):
                wv = w8 + ds
                for gi in range(8):
                    cb[ds, pl.ds(gi * nl, nl)] = plsc.load_gather(
                        rb, [base_vecs[gi], wv])
                return carry

            lax.fori_loop(0, 8, ds_body, 0, unroll=False)
            pltpu.async_copy(cb, out_hbm.at[s, dtg, bt], osem[p])

        def half_a(s, bt):
            def pair(j):
                do_dt(r_a, 2 * j, 2 * j, s, bt, 0)
                do_dt(r_a, 2 * j + 1, 2 * j + 1, s, bt, 1)

            lax.fori_loop(0, (_DTA - 1) // 2,
                          lambda j, c: (pair(j), c)[1], 0, unroll=False)
            do_dt(r_a, _DTA - 1, _DTA - 1, s, bt, 0)

        def half_b(s, bt):
            def pair(j):
                do_dt(r_b, 2 * j, _DTA + 2 * j, s, bt, 1)
                do_dt(r_b, 2 * j + 1, _DTA + 2 * j + 1, s, bt, 0)

            lax.fori_loop(0, _DTB // 2,
                          lambda j, c: (pair(j), c)[1], 0, unroll=False)

        def emit_block(n, not_last):
            h = lax.rem(n, 2)
            s, bt = sbt(n)
            wait_ga(h)
            @pl.when(not_last)
            def _():
                load_idx(1 - h, n + 1)
            half_a(s, bt)
            @pl.when(not_last)
            def _():
                wait_idx(1 - h, n + 1)
                start_ga(1 - h)
            wait_gb(h)
            half_b(s, bt)
            @pl.when(not_last)
            def _():
                start_gb(1 - h)

        load_idx(0, 0)
        wait_idx(0, 0)
        start_ga(0)
        start_gb(0)
        s0, bt0 = sbt(0)
        pltpu.async_copy(chunk[0], out_hbm.at[s0, 0, bt0], osem[0])
        pltpu.async_copy(chunk[1], out_hbm.at[s0, 1, bt0], osem[1])

        def block_body(n, carry):
            emit_block(n, n < bpw - 1)
            return carry

        lax.fori_loop(0, bpw, block_body, 0, unroll=False)

        wait_out(0)
        wait_out(1)

    return gather


def kernel(x, table):
    BN, S = x.shape
    V, D = table.shape
    scaled = _scale_table(table)
    tbl_a = scaled[:, :_CA]
    tbl_b = scaled[:, _CA:]
    xt = x.T.astype(jnp.int32)
    o5 = _make_gather(BN, S, V, D)(xt, tbl_a, tbl_b)
    return jnp.transpose(o5, (2, 4, 0, 1, 3)).reshape(BN, S, D)


# speedup vs baseline: 1.2308x; 1.0102x over previous
import functools
import math

import jax
import jax.numpy as jnp
from jax import lax
from jax.experimental import pallas as pl
from jax.experimental.pallas import tpu as pltpu
from jax.experimental.pallas import tpu_sc as plsc

_SCALE = math.sqrt(512.0)
_BB = 128
_CA = 504
_CB = 496
_DTA = _CA // 8
_DTB = _CB // 8


def _scale_body(t_ref, o_ref):
    o_ref[...] = t_ref[...] * _SCALE


def _scale_table(table):
    return pl.pallas_call(
        _scale_body,
        out_shape=jax.ShapeDtypeStruct(table.shape, table.dtype),
    )(table)


@functools.cache
def _make_gather(BN, S, V, D):
    info = plsc.get_sparse_core_info()
    nc, ns, nl = info.num_cores, info.num_subcores, info.num_lanes
    nw = nc * ns
    n_btiles = BN // _BB
    n_blocks = S * n_btiles
    bpw = n_blocks // nw
    n_bpairs = bpw // 2
    mesh = plsc.VectorSubcoreMesh(core_axis_name="c", subcore_axis_name="s")

    @functools.partial(
        pl.kernel,
        out_type=jax.ShapeDtypeStruct((S, D // 8, n_btiles, 8, _BB),
                                      jnp.float32),
        mesh=mesh,
        scratch_types=[
            pltpu.VMEM((2, _BB), jnp.int32),
            pltpu.VMEM((_BB, _CA), jnp.float32),
            pltpu.VMEM((_BB, _CB), jnp.float32),
            pltpu.VMEM((8, _BB), jnp.float32),
            pltpu.VMEM((8, _BB), jnp.float32),
            pltpu.SemaphoreType.DMA,
            pltpu.SemaphoreType.DMA,
            pltpu.SemaphoreType.DMA,
            pltpu.SemaphoreType.DMA,
            pltpu.SemaphoreType.DMA,
        ],
        compiler_params=pltpu.CompilerParams(use_tc_tiling_on_sc=False,
                                             needs_layout_passes=False,
                                             disable_bounds_checks=True),
    )
    def gather(xt_hbm, tbl_a, tbl_b, out_hbm,
               idx_v, r_a, r_b, c0, c1,
               gs_a, gs_b, os0, os1, isem):
        cid = lax.axis_index("c")
        sid = lax.axis_index("s")
        wid = sid * nc + cid
        g_base = wid * bpw
        chunk = (c0, c1)
        osem = (os0, os1)

        def sbt(blk):
            g = g_base + blk
            return lax.div(g, n_btiles), lax.rem(g, n_btiles)

        def idx_src(blk):
            s, bt = sbt(blk)
            return xt_hbm.at[s, pl.ds(bt * _BB, _BB)]

        def load_idx(p, blk):
            pltpu.async_copy(idx_src(blk), idx_v.at[p], isem)

        def wait_idx(p, blk):
            pltpu.make_async_copy(idx_src(blk), idx_v.at[p], isem).wait()

        def start_ga(p):
            pltpu.async_copy(tbl_a.at[idx_v.at[p]], r_a, gs_a)

        def wait_ga(p):
            pltpu.make_async_copy(tbl_a.at[idx_v.at[p]], r_a, gs_a).wait()

        def start_gb(p):
            pltpu.async_copy(tbl_b.at[idx_v.at[p]], r_b, gs_b)

        def wait_gb(p):
            pltpu.make_async_copy(tbl_b.at[idx_v.at[p]], r_b, gs_b).wait()

        def wait_out(p):
            pltpu.make_async_copy(chunk[p], out_hbm.at[0, 0, 0],
                                  osem[p]).wait()

        base_vecs = [
            (lax.iota(jnp.int32, nl) + g * nl) for g in range(8)
        ]

        def do_dt(rb, d, dtg, s, bt, p):
            wait_out(p)
            cb = chunk[p]
            w8 = jnp.full((nl,), 0, jnp.int32) + d * 8

            def ds_body(dsh, carry):
                for dsl in range(2):
                    ds = 2 * dsh + dsl
                    wv = w8 + ds
                    for gi in range(8):
                        cb[ds, pl.ds(gi * nl, nl)] = plsc.load_gather(
                            rb, [base_vecs[gi], wv])
                return carry

            lax.fori_loop(0, 4, ds_body, 0, unroll=False)
            pltpu.async_copy(cb, out_hbm.at[s, dtg, bt], osem[p])

        def half_a(s, bt):
            def pair(j):
                do_dt(r_a, 2 * j, 2 * j, s, bt, 0)
                do_dt(r_a, 2 * j + 1, 2 * j + 1, s, bt, 1)

            lax.fori_loop(0, (_DTA - 1) // 2,
                          lambda j, c: (pair(j), c)[1], 0, unroll=False)
            do_dt(r_a, _DTA - 1, _DTA - 1, s, bt, 0)

        def half_b(s, bt):
            def pair(j):
                do_dt(r_b, 2 * j, _DTA + 2 * j, s, bt, 1)
                do_dt(r_b, 2 * j + 1, _DTA + 2 * j + 1, s, bt, 0)

            lax.fori_loop(0, _DTB // 2,
                          lambda j, c: (pair(j), c)[1], 0, unroll=False)

        def emit_block(n, not_last):
            h = lax.rem(n, 2)
            s, bt = sbt(n)
            wait_ga(h)
            @pl.when(not_last)
            def _():
                load_idx(1 - h, n + 1)
            half_a(s, bt)
            @pl.when(not_last)
            def _():
                wait_idx(1 - h, n + 1)
                start_ga(1 - h)
            wait_gb(h)
            half_b(s, bt)
            @pl.when(not_last)
            def _():
                start_gb(1 - h)

        load_idx(0, 0)
        wait_idx(0, 0)
        start_ga(0)
        start_gb(0)
        s0, bt0 = sbt(0)
        pltpu.async_copy(chunk[0], out_hbm.at[s0, 0, bt0], osem[0])
        pltpu.async_copy(chunk[1], out_hbm.at[s0, 1, bt0], osem[1])

        def block_body(n, carry):
            emit_block(n, n < bpw - 1)
            return carry

        lax.fori_loop(0, bpw, block_body, 0, unroll=False)

        wait_out(0)
        wait_out(1)

    return gather


def kernel(x, table):
    BN, S = x.shape
    V, D = table.shape
    scaled = _scale_table(table)
    tbl_a = scaled[:, :_CA]
    tbl_b = scaled[:, _CA:]
    xt = x.T.astype(jnp.int32)
    o5 = _make_gather(BN, S, V, D)(xt, tbl_a, tbl_b)
    return jnp.transpose(o5, (2, 4, 0, 1, 3)).reshape(BN, S, D)


# speedup vs baseline: 1.4318x; 1.1633x over previous
import functools
import math

import jax
import jax.numpy as jnp
from jax import lax
from jax.experimental import pallas as pl
from jax.experimental.pallas import tpu as pltpu
from jax.experimental.pallas import tpu_sc as plsc

_SCALE = math.sqrt(512.0)
_BB = 128
_NSL = 5
_CW = 200
_NDT = 25


def _scale_body(t_ref, o_ref):
    o_ref[...] = t_ref[...] * _SCALE


def _scale_table(table):
    return pl.pallas_call(
        _scale_body,
        out_shape=jax.ShapeDtypeStruct(table.shape, table.dtype),
    )(table)


@functools.cache
def _make_gather(BN, S, V, D):
    info = plsc.get_sparse_core_info()
    nc, ns, nl = info.num_cores, info.num_subcores, info.num_lanes
    nw = nc * ns
    n_btiles = BN // _BB
    n_blocks = S * n_btiles
    blocks_per_w = n_blocks // nw
    n_steps = blocks_per_w * _NSL
    n_pairs = n_steps // 10
    mesh = plsc.VectorSubcoreMesh(core_axis_name="c", subcore_axis_name="s")

    @functools.partial(
        pl.kernel,
        out_type=jax.ShapeDtypeStruct((S, D // 8, n_btiles, 8, _BB),
                                      jnp.float32),
        mesh=mesh,
        scratch_types=[
            pltpu.VMEM_SHARED((V, _CW), jnp.float32),
            pltpu.VMEM_SHARED((V, _CW), jnp.float32),
            pltpu.VMEM_SHARED((V, _CW), jnp.float32),
            pltpu.VMEM_SHARED((V, _CW), jnp.float32),
            pltpu.VMEM_SHARED((V, _CW), jnp.float32),
            pltpu.VMEM((_BB,), jnp.int32),
            pltpu.VMEM((_BB,), jnp.int32),
            pltpu.VMEM((_BB, _CW), jnp.float32),
            pltpu.VMEM((_BB, _CW), jnp.float32),
            pltpu.VMEM((8, _BB), jnp.float32),
            pltpu.VMEM((8, _BB), jnp.float32),
            pltpu.SemaphoreType.DMA,
            pltpu.SemaphoreType.DMA,
            pltpu.SemaphoreType.DMA,
            pltpu.SemaphoreType.DMA,
            pltpu.SemaphoreType.DMA,
            pltpu.SemaphoreType.DMA,
        ],
        compiler_params=pltpu.CompilerParams(use_tc_tiling_on_sc=False,
                                             needs_layout_passes=False),
    )
    def gather(xt_hbm, t0, t1, t2, t3, t4, out_hbm,
               ts0, ts1, ts2, ts3, ts4,
               idx0, idx1, r0, r1, c0, c1, g0, g1, o0, o1, i0, i1):
        cid = lax.axis_index("c")
        sid = lax.axis_index("s")
        wid = sid * nc + cid
        g_base = wid * blocks_per_w
        tbls = (t0, t1, t2, t3, t4)
        tsh = (ts0, ts1, ts2, ts3, ts4)
        idx = (idx0, idx1)
        rows = (r0, r1)
        chunk = (c0, c1)
        gsem = (g0, g1)
        osem = (o0, o1)
        isem = (i0, i1)

        def idx_src(blk):
            g = g_base + blk
            s = lax.div(g, n_btiles)
            bt = lax.rem(g, n_btiles)
            return xt_hbm.at[s, pl.ds(bt * _BB, _BB)]

        def load_idx(p, blk):
            pltpu.async_copy(idx_src(blk), idx[p], isem[p])

        def wait_idx(p, blk):
            pltpu.make_async_copy(idx_src(blk), idx[p], isem[p]).wait()

        def start_gather(k, p, b):
            pltpu.async_copy(tsh[k].at[idx[p]], rows[b], gsem[b])

        def wait_gather(k, p, b):
            pltpu.make_async_copy(tsh[k].at[idx[p]], rows[b],
                                  gsem[b]).wait()

        def wait_out(p):
            pltpu.make_async_copy(chunk[p], out_hbm.at[0, 0, 0],
                                  osem[p]).wait()

        base_vecs = [
            (lax.iota(jnp.int32, nl) + g * nl) for g in range(8)
        ]

        def transpose_write_slice(b, blk, k):
            rb = rows[b]
            g = g_base + blk
            s = lax.div(g, n_btiles)
            bt = lax.rem(g, n_btiles)

            def do_dt(dtl, p):
                wait_out(p)
                cb = chunk[p]
                w8 = jnp.full((nl,), 0, jnp.int32) + dtl * 8
                for ds in range(8):
                    wv = w8 + ds
                    for gi in range(8):
                        vals = plsc.load_gather(rb, [base_vecs[gi], wv])
                        cb[ds, pl.ds(gi * nl, nl)] = vals
                pltpu.async_copy(cb, out_hbm.at[s, k * _NDT + dtl, bt],
                                 osem[p])

            def dt_pair(jj, carry):
                do_dt(2 * jj, 0)
                do_dt(2 * jj + 1, 1)
                return carry

            lax.fori_loop(0, _NDT // 2, dt_pair, 0, unroll=False)
            do_dt(_NDT - 1, 0)

        def emit_step(j, u):
            k = u % 5
            b = u % 2
            p = u // 5
            blk = 2 * j + u // 5
            not_first = j > 0
            not_last = j < n_pairs - 1
            if u == 0:
                @pl.when(not_first)
                def _():
                    load_idx(1, 2 * j + 1)
            if u == 5:
                @pl.when(not_last)
                def _():
                    load_idx(0, 2 * j + 2)
            if u == 3:
                @pl.when(not_first)
                def _():
                    wait_idx(1, 2 * j + 1)
            if u == 8:
                @pl.when(not_last)
                def _():
                    wait_idx(0, 2 * j + 2)
            wait_gather(k, p, b)
            transpose_write_slice(b, blk, k)
            if u < 8:
                start_gather((u + 2) % 5, (u + 2) // 5, b)
            else:
                @pl.when(not_last)
                def _():
                    start_gather(u - 8, 0, b)

        @pl.when(sid == 0)
        def _():
            for kk in range(5):
                pltpu.sync_copy(tbls[kk], tsh[kk])

        plsc.subcore_barrier()

        load_idx(0, 0)
        load_idx(1, 1)
        wait_idx(0, 0)
        wait_idx(1, 1)
        start_gather(0, 0, 0)
        start_gather(1, 0, 1)
        g0b = g_base
        s00 = lax.div(g0b, n_btiles)
        bt00 = lax.rem(g0b, n_btiles)
        pltpu.async_copy(chunk[0], out_hbm.at[s00, 0, bt00], osem[0])
        pltpu.async_copy(chunk[1], out_hbm.at[s00, 1, bt00], osem[1])

        def pair_body(j, carry):
            for u in range(10):
                emit_step(j, u)
            return carry

        lax.fori_loop(0, n_pairs, pair_body, 0, unroll=False)

        wait_out(0)
        wait_out(1)

    return gather


def kernel(x, table):
    BN, S = x.shape
    V, D = table.shape
    scaled = _scale_table(table)
    tbls = [scaled[:, k * _CW:(k + 1) * _CW] for k in range(_NSL)]
    xt = x.T.astype(jnp.int32)
    o5 = _make_gather(BN, S, V, D)(xt, *tbls)
    return jnp.transpose(o5, (2, 4, 0, 1, 3)).reshape(BN, S, D)
